# baseline (speedup 1.0000x reference)
import jax
import jax.numpy as jnp
from jax import lax
from jax.experimental import pallas as pl
from jax.experimental.pallas import tpu as pltpu
from jax.experimental.pallas import tpu_sc as plsc

_VOCAB = 1000000
_D = 32
_B = 4096
_L = 200
_NC = 2
_NS = 16
_NW = _NC * _NS
_TB = _B // _NW


def _gather_kernel(idx_hbm, table_hbm, out_hbm,
                   idx_v, col0, col1, rows0, rows1, tr0, tr1,
                   gsem0, gsem1, ssem0, ssem1):
    wid = lax.axis_index("s") * _NC + lax.axis_index("c")

    cols = (col0, col1)
    rows = (rows0, rows1)
    trs = (tr0, tr1)
    gsems = (gsem0, gsem1)
    ssems = (ssem0, ssem1)

    pltpu.sync_copy(idx_hbm.at[pl.ds(wid * _TB, _TB)], idx_v)

    tok16 = lax.iota(jnp.int32, 16)

    def fire(b, l):
        for g in range(_TB // 16):
            t_vec = tok16 + (16 * g)
            vals = plsc.load_gather(idx_v, [t_vec, jnp.full((16,), l, jnp.int32)])
            cols[b][pl.ds(16 * g, 16)] = vals
        pltpu.async_copy(table_hbm.at[cols[b]], rows[b], gsems[b])

    def drain_stores(b):
        for r in range(4):
            pltpu.make_async_copy(
                out_hbm.at[0, 0, 0], trs[b].at[pl.ds(8 * r, 8)],
                ssems[b]).wait()

    def process(b, l, i):
        pltpu.make_async_copy(
            table_hbm.at[pl.ds(0, _TB)], rows[b], gsems[b]).wait()

        @pl.when(i > 0)
        def _():
            drain_stores(b)

        for t in range(_TB // 16):
            t_vec = tok16 + (16 * t)
            for d in range(_D):
                vals = plsc.load_gather(
                    rows[b], [t_vec, jnp.full((16,), d, jnp.int32)])
                trs[b][d, pl.ds(16 * t, 16)] = vals

        for r in range(4):
            pltpu.async_copy(
                trs[b].at[pl.ds(8 * r, 8)],
                out_hbm.at[l, r, wid],
                ssems[b],
            )

    fire(0, 0)
    fire(1, 1)

    @pl.loop(0, _L // 2)
    def body(i):
        l0 = 2 * i
        process(0, l0, i)

        @pl.when(i < _L // 2 - 1)
        def _():
            fire(0, l0 + 2)

        process(1, l0 + 1, i)

        @pl.when(i < _L // 2 - 1)
        def _():
            fire(1, l0 + 3)

    drain_stores(0)
    drain_stores(1)


@jax.jit
def _embed_lookup(indices, table):
    mesh = plsc.VectorSubcoreMesh(core_axis_name="c", subcore_axis_name="s")
    out5 = pl.kernel(
        _gather_kernel,
        out_type=jax.ShapeDtypeStruct((_L, 4, _NW, 8, 128), jnp.float32),
        mesh=mesh,
        scratch_types=[
            pltpu.VMEM((_TB, _L), jnp.int32),
            pltpu.VMEM((_TB,), jnp.int32),
            pltpu.VMEM((_TB,), jnp.int32),
            pltpu.VMEM((_TB, _D), jnp.float32),
            pltpu.VMEM((_TB, _D), jnp.float32),
            pltpu.VMEM((_D, _TB), jnp.float32),
            pltpu.VMEM((_D, _TB), jnp.float32),
            pltpu.SemaphoreType.DMA,
            pltpu.SemaphoreType.DMA,
            pltpu.SemaphoreType.DMA,
            pltpu.SemaphoreType.DMA,
        ],
        compiler_params=pltpu.CompilerParams(
            use_tc_tiling_on_sc=False, needs_layout_passes=False),
    )(indices, table)
    return out5.transpose(2, 4, 0, 1, 3).reshape(_B, _L, _D)


def kernel(indices, table):
    return _embed_lookup(indices, table)


# speedup vs baseline: 1.1378x; 1.1378x over previous
import jax
import jax.numpy as jnp
from jax import lax
from jax.experimental import pallas as pl
from jax.experimental.pallas import tpu as pltpu
from jax.experimental.pallas import tpu_sc as plsc

_VOCAB = 1000000
_D = 32
_B = 4096
_L = 200
_NC = 2
_NS = 16
_NW = _NC * _NS
_TB = _B // _NW


def _gather_kernel(idx_hbm, table_hbm, out_hbm,
                   idx_v, col0, col1, rows0, rows1, tr0, tr1,
                   gsem0, gsem1, ssem0, ssem1):
    wid = lax.axis_index("s") * _NC + lax.axis_index("c")

    cols = (col0, col1)
    rows = (rows0, rows1)
    trs = (tr0, tr1)
    gsems = (gsem0, gsem1)
    ssems = (ssem0, ssem1)

    pltpu.sync_copy(idx_hbm.at[pl.ds(wid * _TB, _TB)], idx_v)

    tok16 = lax.iota(jnp.int32, 16)
    d_lo = tok16
    d_hi = tok16 + 16

    def fire(b, l):
        for g in range(_TB // 16):
            t_vec = tok16 + (16 * g)
            vals = plsc.load_gather(
                idx_v, [t_vec, jnp.full((16,), l, jnp.int32)])
            cols[b][pl.ds(16 * g, 16)] = vals
        pltpu.async_copy(table_hbm.at[cols[b]], rows[b], gsems[b])

    def drain_gather(b):
        pltpu.make_async_copy(
            table_hbm.at[pl.ds(0, _TB)], rows[b], gsems[b]).wait()

    def drain_stores(b):
        for r in range(4):
            pltpu.make_async_copy(
                out_hbm.at[0, 0, 0], trs[b].at[pl.ds(8 * r, 8)],
                ssems[b]).wait()

    def process(b, l, i):
        drain_gather(b)

        @pl.when(i > 0)
        def _():
            drain_stores(b)

        for t in range(_TB):
            lo = rows[b][t, pl.ds(0, 16)]
            hi = rows[b][t, pl.ds(16, 16)]
            t_vec = jnp.full((16,), t, jnp.int32)
            plsc.store_scatter(trs[b], [d_lo, t_vec], lo)
            plsc.store_scatter(trs[b], [d_hi, t_vec], hi)

        for r in range(4):
            pltpu.async_copy(
                trs[b].at[pl.ds(8 * r, 8)],
                out_hbm.at[l, r, wid],
                ssems[b],
            )

    fire(0, 0)
    fire(1, 1)

    @pl.loop(0, _L // 2)
    def body(i):
        l0 = 2 * i
        process(0, l0, i)

        @pl.when(i < _L // 2 - 1)
        def _():
            fire(0, l0 + 2)

        process(1, l0 + 1, i)

        @pl.when(i < _L // 2 - 1)
        def _():
            fire(1, l0 + 3)

    drain_stores(0)
    drain_stores(1)


@jax.jit
def _embed_lookup(indices, table):
    mesh = plsc.VectorSubcoreMesh(core_axis_name="c", subcore_axis_name="s")
    out5 = pl.kernel(
        _gather_kernel,
        out_type=jax.ShapeDtypeStruct((_L, 4, _NW, 8, 128), jnp.float32),
        mesh=mesh,
        scratch_types=[
            pltpu.VMEM((_TB, _L), jnp.int32),
            pltpu.VMEM((_TB,), jnp.int32),
            pltpu.VMEM((_TB,), jnp.int32),
            pltpu.VMEM((_TB, _D), jnp.float32),
            pltpu.VMEM((_TB, _D), jnp.float32),
            pltpu.VMEM((_D, _TB), jnp.float32),
            pltpu.VMEM((_D, _TB), jnp.float32),
            pltpu.SemaphoreType.DMA,
            pltpu.SemaphoreType.DMA,
            pltpu.SemaphoreType.DMA,
            pltpu.SemaphoreType.DMA,
        ],
        compiler_params=pltpu.CompilerParams(
            use_tc_tiling_on_sc=False, needs_layout_passes=False),
    )(indices, table)
    return out5.transpose(2, 4, 0, 1, 3).reshape(_B, _L, _D)


def kernel(indices, table):
    return _embed_lookup(indices, table)
